# baseline (speedup 1.0000x reference)
import functools

import jax
import jax.numpy as jnp
from jax import lax
from jax.experimental import pallas as pl
from jax.experimental.pallas import tpu as pltpu
from jax.experimental.pallas import tpu_sc as plsc

KNN = 32
N_REAL = 10000
NW = 32
NC, NS, NL = 2, 16, 16
NP = 10240
SENT = 10000
NPW = NP // NW
EPW = NPW * KNN
BN = 4
NB = NPW // BN
QB = 256
RB = 512
PADC = 1.0e18
BIGF = 3.0e38
INV_SQRT_K = float(1.0 / (32.0 ** 0.5))
F32 = jnp.float32
I32 = jnp.int32

@functools.cache
def _mesh():
    return plsc.VectorSubcoreMesh(core_axis_name="c", subcore_axis_name="s",
                                  num_cores=NC, num_subcores=NS)


CH = NP // 128
NSLOT = 4
RG = 16
BIGI = 2**30


def _lex_lt(av, ai, bv, bi):
    return (av < bv) | ((av == bv) & (ai < bi))


def _knn_body(cand3_ref, pts_ref, out_ref):
    i = pl.program_id(0)
    qx = pts_ref[:, 0:1]
    qy = pts_ref[:, 1:2]
    qz = pts_ref[:, 2:3]
    qsq = (qx * qx + qy * qy) + qz * qz
    lane = lax.broadcasted_iota(I32, (RG, 128), 1)
    bigv = jnp.full((RG, 128), BIGF, F32)
    bigi = jnp.full((RG, 128), BIGI, I32)
    outcol = lax.broadcasted_iota(I32, (RG, KNN), 1)

    def _chunk_d(c):
        cx = cand3_ref[0, pl.ds(c, 1), :]
        cy = cand3_ref[1, pl.ds(c, 1), :]
        cz = cand3_ref[2, pl.ds(c, 1), :]
        csq = (cx * cx + cy * cy) + cz * cz
        d = (qsq + csq) - 2.0 * (qx * cx + qy * cy + qz * cz)
        return d, lane + c * 128

    def _insert(vs, ids, nv, ni):
        lt = [nv < vs[s] for s in range(NSLOT)]
        ovs = [jnp.where(lt[0], nv, vs[0])]
        ois = [jnp.where(lt[0], ni, ids[0])]
        for s in range(1, NSLOT):
            ovs.append(jnp.where(lt[s],
                                 jnp.where(lt[s - 1], vs[s - 1], nv), vs[s]))
            ois.append(jnp.where(lt[s],
                                 jnp.where(lt[s - 1], ids[s - 1], ni), ids[s]))
        return tuple(ovs), tuple(ois)

    def build(c, st):
        vs, ids = st
        nv0, ni0 = _chunk_d(c)
        return _insert(vs, ids, nv0, ni0)

    vs, ids = lax.fori_loop(0, CH, build,
                            (tuple([bigv] * NSLOT), tuple([bigi] * NSLOT)))

    def ext(t, st):
        outb, lastv, lasti = st
        cv, ci = bigv, bigi
        for s in range(NSLOT):
            ok = _lex_lt(lastv, lasti, vs[s], ids[s])
            sv = jnp.where(ok, vs[s], BIGF)
            si = jnp.where(ok, ids[s], BIGI)
            t2 = _lex_lt(sv, si, cv, ci)
            cv = jnp.where(t2, sv, cv)
            ci = jnp.where(t2, si, ci)
        mv = jnp.min(cv, axis=1, keepdims=True)
        mi = jnp.min(jnp.where(cv == mv, ci, BIGI), axis=1, keepdims=True)
        outb = jnp.where(outcol == t, jnp.broadcast_to(mi, (RG, KNN)), outb)
        return outb, mv, mi

    init = (jnp.zeros((RG, KNN), I32),
            jnp.full((RG, 1), -BIGF, F32), jnp.full((RG, 1), -1, I32))
    outb, lastv, lasti = lax.fori_loop(0, KNN, ext, init)

    s7v, s7i = vs[NSLOT - 1], ids[NSLOT - 1]
    bad = jnp.any((s7v < lastv) | ((s7v == lastv) & (s7i <= lasti)))

    def slow(_):
        def ext1(t, st):
            outb, lastv, lasti = st

            def scan(c, st2):
                cv, ci = st2
                d, gi = _chunk_d(c)
                ok = _lex_lt(lastv, lasti, d, gi)
                sv = jnp.where(ok, d, BIGF)
                si = jnp.where(ok, gi, BIGI)
                t2 = _lex_lt(sv, si, cv, ci)
                return jnp.where(t2, sv, cv), jnp.where(t2, si, ci)

            cv, ci = lax.fori_loop(0, CH, scan, (bigv, bigi))
            mv = jnp.min(cv, axis=1, keepdims=True)
            mi = jnp.min(jnp.where(cv == mv, ci, BIGI), axis=1, keepdims=True)
            outb = jnp.where(outcol == t, jnp.broadcast_to(mi, (RG, KNN)), outb)
            return outb, mv, mi

        outb, _, _ = lax.fori_loop(0, KNN, ext1, init)
        return outb

    outb = lax.cond(bad, slow, lambda _: outb, 0)
    row = i * RG + lax.broadcasted_iota(I32, (RG, 1), 0)
    out_ref[...] = jnp.where(row < N_REAL, outb, SENT)


def _knn_topk(cand3, pts_pad):
    return pl.pallas_call(
        _knn_body,
        grid=(NP // RG,),
        in_specs=[
            pl.BlockSpec((3, CH, 128), lambda i: (0, 0, 0)),
            pl.BlockSpec((RG, 3), lambda i: (i, 0)),
        ],
        out_specs=pl.BlockSpec((RG, KNN), lambda i: (i, 0)),
        out_shape=jax.ShapeDtypeStruct((NP, KNN), I32),
    )(cand3, pts_pad)


@functools.cache
def _sc_hist_fn():
    @functools.partial(
        pl.kernel,
        mesh=_mesh(),
        out_type=jax.ShapeDtypeStruct((NW, NP), F32),
        scratch_types=[
            pltpu.VMEM((EPW,), I32),
            pltpu.VMEM((NP,), F32),
        ],
        compiler_params=pltpu.CompilerParams(needs_layout_passes=False),
    )
    def _sc_hist(idx_hbm, out_hbm, idx_v, hist_v):
        wid = lax.axis_index("s") * NC + lax.axis_index("c")
        pltpu.sync_copy(idx_hbm.at[wid], idx_v)
        zero16 = jnp.zeros((NL,), F32)
        one16 = jnp.full((NL,), 1.0, F32)

        def zbody(j, _):
            hist_v[pl.ds(j * NL, NL)] = zero16
            return 0

        lax.fori_loop(0, NP // NL, zbody, 0)

        def abody(j, _):
            v = idx_v[pl.ds(j * NL, NL)]
            plsc.addupdate_scatter(hist_v, [v], one16)
            return 0

        lax.fori_loop(0, EPW // NL, abody, 0)
        pltpu.sync_copy(hist_v, out_hbm.at[wid])

    return _sc_hist


def _deg_body(h_ref, o_ref):
    s = jnp.sum(h_ref[...], axis=0, keepdims=True)
    o_ref[...] = lax.rsqrt(jnp.maximum(s, 1.0))


def _deg_w(hist):
    return pl.pallas_call(
        _deg_body,
        out_shape=jax.ShapeDtypeStruct((1, NP), F32),
    )(hist)


def _scale_body(x_ref, w_ref, o_ref):
    o_ref[...] = x_ref[...] * w_ref[...]


def _scale_rows(x_pad, wcol):
    return pl.pallas_call(
        _scale_body,
        grid=(NP // RB,),
        in_specs=[
            pl.BlockSpec((RB, 128), lambda i: (i, 0)),
            pl.BlockSpec((RB, 1), lambda i: (i, 0)),
        ],
        out_specs=pl.BlockSpec((RB, 128), lambda i: (i, 0)),
        out_shape=jax.ShapeDtypeStruct((NP, 128), F32),
    )(x_pad, wcol)


def _tree_sum(vs):
    while len(vs) > 1:
        nxt = [vs[i] + vs[i + 1] for i in range(0, len(vs) - 1, 2)]
        if len(vs) % 2:
            nxt.append(vs[-1])
        vs = nxt
    return vs[0]


@functools.cache
def _sc_gather_fn():
    @functools.partial(
        pl.kernel,
        mesh=_mesh(),
        out_type=jax.ShapeDtypeStruct((NP, 128), F32),
        scratch_types=[
            pltpu.VMEM((NB, BN * KNN), I32),
            pltpu.VMEM((BN * KNN, 128), F32),
            pltpu.VMEM((BN, 128), F32),
            pltpu.SemaphoreType.DMA,
        ],
        compiler_params=pltpu.CompilerParams(needs_layout_passes=False),
    )
    def _sc_gather(hs_hbm, idx_hbm, out_hbm, idx_v, rows_v, out_v, sem):
        wid = lax.axis_index("s") * NC + lax.axis_index("c")
        pltpu.sync_copy(idx_hbm.at[wid], idx_v)

        def body(b, _):
            pltpu.async_copy(hs_hbm.at[idx_v.at[b]], rows_v, sem).wait()
            for n in range(BN):
                for c in range(8):
                    sl = pl.ds(c * NL, NL)
                    acc = _tree_sum([rows_v[n * KNN + m, sl] for m in range(KNN)])
                    out_v[n, sl] = acc
            pltpu.sync_copy(out_v, out_hbm.at[pl.ds(wid * NPW + b * BN, BN)])
            return 0

        lax.fori_loop(0, NB, body, 0)

    return _sc_gather


def _mm_body(a_ref, w_ref, b_ref, wc_ref, o_ref):
    a = a_ref[...] * INV_SQRT_K
    m = lax.dot_general(a, w_ref[...], (((1,), (0,)), ((), ())),
                        precision=lax.Precision.HIGHEST,
                        preferred_element_type=F32)
    h = jnp.maximum(m + b_ref[...], 0.0)
    o_ref[...] = h * wc_ref[...]


def _mm_layer(agg, W, b, wcol):
    return pl.pallas_call(
        _mm_body,
        grid=(NP // RB,),
        in_specs=[
            pl.BlockSpec((RB, 128), lambda i: (i, 0)),
            pl.BlockSpec((128, 128), lambda i: (0, 0)),
            pl.BlockSpec((1, 128), lambda i: (0, 0)),
            pl.BlockSpec((RB, 1), lambda i: (i, 0)),
        ],
        out_specs=pl.BlockSpec((RB, 128), lambda i: (i, 0)),
        out_shape=jax.ShapeDtypeStruct((NP, 128), F32),
    )(agg, W, b, wcol)


def _mm_final_body(a_ref, w_ref, b_ref, wf_ref, bf_ref, o_ref):
    a = a_ref[...] * INV_SQRT_K
    m = lax.dot_general(a, w_ref[...], (((1,), (0,)), ((), ())),
                        precision=lax.Precision.HIGHEST,
                        preferred_element_type=F32)
    h = jnp.maximum(m + b_ref[...], 0.0)
    z = lax.dot_general(h, wf_ref[...], (((1,), (0,)), ((), ())),
                        precision=lax.Precision.HIGHEST,
                        preferred_element_type=F32) + bf_ref[...]
    o_ref[...] = 1.0 / (1.0 + jnp.exp(-z))


def _mm_final(agg, W, b, Wf, bf):
    return pl.pallas_call(
        _mm_final_body,
        grid=(NP // RB,),
        in_specs=[
            pl.BlockSpec((RB, 128), lambda i: (i, 0)),
            pl.BlockSpec((128, 128), lambda i: (0, 0)),
            pl.BlockSpec((1, 128), lambda i: (0, 0)),
            pl.BlockSpec((128, 1), lambda i: (0, 0)),
            pl.BlockSpec((1, 1), lambda i: (0, 0)),
        ],
        out_specs=pl.BlockSpec((RB, 1), lambda i: (i, 0)),
        out_shape=jax.ShapeDtypeStruct((NP, 1), F32),
    )(agg, W, b, Wf, bf)


def kernel(x, triangle_centers, W1, b1, W2, b2, W3, b3, Wf, bf):
    pts = triangle_centers
    pad_n = NP - N_REAL
    pts_pad = jnp.pad(pts, ((0, pad_n), (0, 0)), constant_values=PADC)
    cand3 = jnp.pad(pts.T, ((0, 0), (0, pad_n)),
                    constant_values=PADC).reshape(3, CH, 128)
    x_pad = jnp.pad(x, ((0, pad_n), (0, 0)))

    idx = _knn_topk(cand3, pts_pad)
    hist = _sc_hist_fn()(idx.reshape(NW, EPW))
    w1d = _deg_w(hist)
    wcol = w1d.reshape(NP, 1)

    idx3 = idx.reshape(NW, NB, BN * KNN)
    h = _scale_rows(x_pad, wcol)
    for W, b in ((W1, b1), (W2, b2)):
        agg = _sc_gather_fn()(h, idx3)
        h = _mm_layer(agg, W, b.reshape(1, 128), wcol)
    agg = _sc_gather_fn()(h, idx3)
    res = _mm_final(agg, W3, b3.reshape(1, 128), Wf, bf.reshape(1, 1))
    return res[:N_REAL, 0]


# speedup vs baseline: 4.9577x; 4.9577x over previous
import functools

import jax
import jax.numpy as jnp
from jax import lax
from jax.experimental import pallas as pl
from jax.experimental.pallas import tpu as pltpu
from jax.experimental.pallas import tpu_sc as plsc

KNN = 32
N_REAL = 10000
NW = 32
NC, NS, NL = 2, 16, 16
NP = 10240
SENT = 10000
NPW = NP // NW
EPW = NPW * KNN
BN = 4
NB = NPW // BN
QB = 256
RB = 512
PADC = 1.0e18
BIGF = 3.0e38
INV_SQRT_K = float(1.0 / (32.0 ** 0.5))
F32 = jnp.float32
I32 = jnp.int32

@functools.cache
def _mesh():
    return plsc.VectorSubcoreMesh(core_axis_name="c", subcore_axis_name="s",
                                  num_cores=NC, num_subcores=NS)


CH = NP // 128
NSLOT = 6
RG = 16
BIGI = 2**30


def _lex_lt(av, ai, bv, bi):
    return (av < bv) | ((av == bv) & (ai < bi))


def _knn_body(cand3_ref, pts_ref, out_ref):
    i = pl.program_id(0)
    qx = pts_ref[:, 0:1]
    qy = pts_ref[:, 1:2]
    qz = pts_ref[:, 2:3]
    qsq = (qx * qx + qy * qy) + qz * qz
    lane = lax.broadcasted_iota(I32, (RG, 128), 1)
    bigv = jnp.full((RG, 128), BIGF, F32)
    bigi = jnp.full((RG, 128), BIGI, I32)
    outcol = lax.broadcasted_iota(I32, (RG, KNN), 1)

    def _chunk_d(c):
        cx = cand3_ref[0, pl.ds(c, 1), :]
        cy = cand3_ref[1, pl.ds(c, 1), :]
        cz = cand3_ref[2, pl.ds(c, 1), :]
        csq = (cx * cx + cy * cy) + cz * cz
        d = (qsq + csq) - 2.0 * (qx * cx + qy * cy + qz * cz)
        return d, lane + c * 128

    def _chunk_d_static(c):
        cx = cand3_ref[0, c, :].reshape(1, 128)
        cy = cand3_ref[1, c, :].reshape(1, 128)
        cz = cand3_ref[2, c, :].reshape(1, 128)
        csq = (cx * cx + cy * cy) + cz * cz
        d = (qsq + csq) - 2.0 * (qx * cx + qy * cy + qz * cz)
        return d, lane + c * 128

    def _insert(vs, ids, nv, ni):
        lt = [nv < vs[s] for s in range(NSLOT)]
        ovs = [jnp.where(lt[0], nv, vs[0])]
        ois = [jnp.where(lt[0], ni, ids[0])]
        for s in range(1, NSLOT):
            ovs.append(jnp.where(lt[s],
                                 jnp.where(lt[s - 1], vs[s - 1], nv), vs[s]))
            ois.append(jnp.where(lt[s],
                                 jnp.where(lt[s - 1], ids[s - 1], ni), ids[s]))
        return tuple(ovs), tuple(ois)

    vs, ids = tuple([bigv] * NSLOT), tuple([bigi] * NSLOT)
    for c in range(CH):
        nv0, ni0 = _chunk_d_static(c)
        vs, ids = _insert(vs, ids, nv0, ni0)

    outb = jnp.zeros((RG, KNN), I32)
    lastv = jnp.full((RG, 1), -BIGF, F32)
    lasti = jnp.full((RG, 1), -1, I32)
    for t in range(KNN):
        cv, ci = bigv, bigi
        for s in range(NSLOT):
            ok = _lex_lt(lastv, lasti, vs[s], ids[s])
            sv = jnp.where(ok, vs[s], BIGF)
            si = jnp.where(ok, ids[s], BIGI)
            t2 = _lex_lt(sv, si, cv, ci)
            cv = jnp.where(t2, sv, cv)
            ci = jnp.where(t2, si, ci)
        mv = jnp.min(cv, axis=1, keepdims=True)
        mi = jnp.min(jnp.where(cv == mv, ci, BIGI), axis=1, keepdims=True)
        outb = jnp.where(outcol == t, jnp.broadcast_to(mi, (RG, KNN)), outb)
        lastv, lasti = mv, mi

    init = (jnp.zeros((RG, KNN), I32),
            jnp.full((RG, 1), -BIGF, F32), jnp.full((RG, 1), -1, I32))

    s7v, s7i = vs[NSLOT - 1], ids[NSLOT - 1]
    bad = jnp.any((s7v < lastv) | ((s7v == lastv) & (s7i <= lasti)))

    def slow(_):
        def ext1(t, st):
            outb, lastv, lasti = st

            def scan(c, st2):
                cv, ci = st2
                d, gi = _chunk_d(c)
                ok = _lex_lt(lastv, lasti, d, gi)
                sv = jnp.where(ok, d, BIGF)
                si = jnp.where(ok, gi, BIGI)
                t2 = _lex_lt(sv, si, cv, ci)
                return jnp.where(t2, sv, cv), jnp.where(t2, si, ci)

            cv, ci = lax.fori_loop(0, CH, scan, (bigv, bigi))
            mv = jnp.min(cv, axis=1, keepdims=True)
            mi = jnp.min(jnp.where(cv == mv, ci, BIGI), axis=1, keepdims=True)
            outb = jnp.where(outcol == t, jnp.broadcast_to(mi, (RG, KNN)), outb)
            return outb, mv, mi

        outb, _, _ = lax.fori_loop(0, KNN, ext1, init)
        return outb

    outb = lax.cond(bad, slow, lambda _: outb, 0)
    row = i * RG + lax.broadcasted_iota(I32, (RG, 1), 0)
    out_ref[...] = jnp.where(row < N_REAL, outb, SENT)


def _knn_topk(cand3, pts_pad):
    return pl.pallas_call(
        _knn_body,
        grid=(NP // RG,),
        in_specs=[
            pl.BlockSpec((3, CH, 128), lambda i: (0, 0, 0)),
            pl.BlockSpec((RG, 3), lambda i: (i, 0)),
        ],
        out_specs=pl.BlockSpec((RG, KNN), lambda i: (i, 0)),
        out_shape=jax.ShapeDtypeStruct((NP, KNN), I32),
    )(cand3, pts_pad)


@functools.cache
def _sc_hist_fn():
    @functools.partial(
        pl.kernel,
        mesh=_mesh(),
        out_type=jax.ShapeDtypeStruct((NW, NP), F32),
        scratch_types=[
            pltpu.VMEM((EPW,), I32),
            pltpu.VMEM((NP,), F32),
        ],
        compiler_params=pltpu.CompilerParams(needs_layout_passes=False),
    )
    def _sc_hist(idx_hbm, out_hbm, idx_v, hist_v):
        wid = lax.axis_index("s") * NC + lax.axis_index("c")
        pltpu.sync_copy(idx_hbm.at[wid], idx_v)
        zero16 = jnp.zeros((NL,), F32)
        one16 = jnp.full((NL,), 1.0, F32)

        def zbody(j, _):
            hist_v[pl.ds(j * NL, NL)] = zero16
            return 0

        lax.fori_loop(0, NP // NL, zbody, 0)

        def abody(j, _):
            v = idx_v[pl.ds(j * NL, NL)]
            plsc.addupdate_scatter(hist_v, [v], one16)
            return 0

        lax.fori_loop(0, EPW // NL, abody, 0)
        pltpu.sync_copy(hist_v, out_hbm.at[wid])

    return _sc_hist


def _deg_body(h_ref, o_ref):
    s = jnp.sum(h_ref[...], axis=0, keepdims=True)
    o_ref[...] = lax.rsqrt(jnp.maximum(s, 1.0))


def _deg_w(hist):
    return pl.pallas_call(
        _deg_body,
        out_shape=jax.ShapeDtypeStruct((1, NP), F32),
    )(hist)


def _scale_body(x_ref, w_ref, o_ref):
    o_ref[...] = x_ref[...] * w_ref[...]


def _scale_rows(x_pad, wcol):
    return pl.pallas_call(
        _scale_body,
        grid=(NP // RB,),
        in_specs=[
            pl.BlockSpec((RB, 128), lambda i: (i, 0)),
            pl.BlockSpec((RB, 1), lambda i: (i, 0)),
        ],
        out_specs=pl.BlockSpec((RB, 128), lambda i: (i, 0)),
        out_shape=jax.ShapeDtypeStruct((NP, 128), F32),
    )(x_pad, wcol)


def _tree_sum(vs):
    while len(vs) > 1:
        nxt = [vs[i] + vs[i + 1] for i in range(0, len(vs) - 1, 2)]
        if len(vs) % 2:
            nxt.append(vs[-1])
        vs = nxt
    return vs[0]


@functools.cache
def _sc_gather_fn():
    @functools.partial(
        pl.kernel,
        mesh=_mesh(),
        out_type=jax.ShapeDtypeStruct((NP, 128), F32),
        scratch_types=[
            pltpu.VMEM((NB, BN * KNN), I32),
            pltpu.VMEM((BN * KNN, 128), F32),
            pltpu.VMEM((BN, 128), F32),
            pltpu.SemaphoreType.DMA,
        ],
        compiler_params=pltpu.CompilerParams(needs_layout_passes=False),
    )
    def _sc_gather(hs_hbm, idx_hbm, out_hbm, idx_v, rows_v, out_v, sem):
        wid = lax.axis_index("s") * NC + lax.axis_index("c")
        pltpu.sync_copy(idx_hbm.at[wid], idx_v)

        def body(b, _):
            pltpu.async_copy(hs_hbm.at[idx_v.at[b]], rows_v, sem).wait()
            for n in range(BN):
                for c in range(8):
                    sl = pl.ds(c * NL, NL)
                    acc = _tree_sum([rows_v[n * KNN + m, sl] for m in range(KNN)])
                    out_v[n, sl] = acc
            pltpu.sync_copy(out_v, out_hbm.at[pl.ds(wid * NPW + b * BN, BN)])
            return 0

        lax.fori_loop(0, NB, body, 0)

    return _sc_gather


def _mm_body(a_ref, w_ref, b_ref, wc_ref, o_ref):
    a = a_ref[...] * INV_SQRT_K
    m = lax.dot_general(a, w_ref[...], (((1,), (0,)), ((), ())),
                        precision=lax.Precision.HIGHEST,
                        preferred_element_type=F32)
    h = jnp.maximum(m + b_ref[...], 0.0)
    o_ref[...] = h * wc_ref[...]


def _mm_layer(agg, W, b, wcol):
    return pl.pallas_call(
        _mm_body,
        grid=(NP // RB,),
        in_specs=[
            pl.BlockSpec((RB, 128), lambda i: (i, 0)),
            pl.BlockSpec((128, 128), lambda i: (0, 0)),
            pl.BlockSpec((1, 128), lambda i: (0, 0)),
            pl.BlockSpec((RB, 1), lambda i: (i, 0)),
        ],
        out_specs=pl.BlockSpec((RB, 128), lambda i: (i, 0)),
        out_shape=jax.ShapeDtypeStruct((NP, 128), F32),
    )(agg, W, b, wcol)


def _mm_final_body(a_ref, w_ref, b_ref, wf_ref, bf_ref, o_ref):
    a = a_ref[...] * INV_SQRT_K
    m = lax.dot_general(a, w_ref[...], (((1,), (0,)), ((), ())),
                        precision=lax.Precision.HIGHEST,
                        preferred_element_type=F32)
    h = jnp.maximum(m + b_ref[...], 0.0)
    z = lax.dot_general(h, wf_ref[...], (((1,), (0,)), ((), ())),
                        precision=lax.Precision.HIGHEST,
                        preferred_element_type=F32) + bf_ref[...]
    o_ref[...] = 1.0 / (1.0 + jnp.exp(-z))


def _mm_final(agg, W, b, Wf, bf):
    return pl.pallas_call(
        _mm_final_body,
        grid=(NP // RB,),
        in_specs=[
            pl.BlockSpec((RB, 128), lambda i: (i, 0)),
            pl.BlockSpec((128, 128), lambda i: (0, 0)),
            pl.BlockSpec((1, 128), lambda i: (0, 0)),
            pl.BlockSpec((128, 1), lambda i: (0, 0)),
            pl.BlockSpec((1, 1), lambda i: (0, 0)),
        ],
        out_specs=pl.BlockSpec((RB, 1), lambda i: (i, 0)),
        out_shape=jax.ShapeDtypeStruct((NP, 1), F32),
    )(agg, W, b, Wf, bf)


def kernel(x, triangle_centers, W1, b1, W2, b2, W3, b3, Wf, bf):
    pts = triangle_centers
    pad_n = NP - N_REAL
    pts_pad = jnp.pad(pts, ((0, pad_n), (0, 0)), constant_values=PADC)
    cand3 = jnp.pad(pts.T, ((0, 0), (0, pad_n)),
                    constant_values=PADC).reshape(3, CH, 128)
    x_pad = jnp.pad(x, ((0, pad_n), (0, 0)))

    idx = _knn_topk(cand3, pts_pad)
    hist = _sc_hist_fn()(idx.reshape(NW, EPW))
    w1d = _deg_w(hist)
    wcol = w1d.reshape(NP, 1)

    idx3 = idx.reshape(NW, NB, BN * KNN)
    h = _scale_rows(x_pad, wcol)
    for W, b in ((W1, b1), (W2, b2)):
        agg = _sc_gather_fn()(h, idx3)
        h = _mm_layer(agg, W, b.reshape(1, 128), wcol)
    agg = _sc_gather_fn()(h, idx3)
    res = _mm_final(agg, W3, b3.reshape(1, 128), Wf, bf.reshape(1, 1))
    return res[:N_REAL, 0]


# speedup vs baseline: 7.5626x; 1.5254x over previous
import functools

import jax
import jax.numpy as jnp
from jax import lax
from jax.experimental import pallas as pl
from jax.experimental.pallas import tpu as pltpu
from jax.experimental.pallas import tpu_sc as plsc

KNN = 32
N_REAL = 10000
NW = 32
NC, NS, NL = 2, 16, 16
NP = 10240
SENT = 10000
NPW = NP // NW
EPW = NPW * KNN
BN = 4
NB = NPW // BN
QB = 256
RB = 512
PADC = 1.0e18
BIGF = 3.0e38
INV_SQRT_K = float(1.0 / (32.0 ** 0.5))
F32 = jnp.float32
I32 = jnp.int32

@functools.cache
def _mesh():
    return plsc.VectorSubcoreMesh(core_axis_name="c", subcore_axis_name="s",
                                  num_cores=NC, num_subcores=NS)


CH = NP // 128
NSLOT = 6
RG = 32
BIGI = 2**30


def _lex_lt(av, ai, bv, bi):
    return (av < bv) | ((av == bv) & (ai < bi))


def _knn_body(cand3_ref, pts_ref, out_ref):
    i = pl.program_id(0)
    qx = pts_ref[:, 0:1]
    qy = pts_ref[:, 1:2]
    qz = pts_ref[:, 2:3]
    qsq = (qx * qx + qy * qy) + qz * qz
    lane = lax.broadcasted_iota(I32, (RG, 128), 1)
    bigv = jnp.full((RG, 128), BIGF, F32)
    bigi = jnp.full((RG, 128), BIGI, I32)
    outcol = lax.broadcasted_iota(I32, (RG, KNN), 1)

    def _chunk_d(c):
        cx = cand3_ref[0, pl.ds(c, 1), :]
        cy = cand3_ref[1, pl.ds(c, 1), :]
        cz = cand3_ref[2, pl.ds(c, 1), :]
        csq = (cx * cx + cy * cy) + cz * cz
        d = (qsq + csq) - 2.0 * (qx * cx + qy * cy + qz * cz)
        return d, lane + c * 128

    def _chunk_d_static(c):
        cx = cand3_ref[0, c, :].reshape(1, 128)
        cy = cand3_ref[1, c, :].reshape(1, 128)
        cz = cand3_ref[2, c, :].reshape(1, 128)
        csq = (cx * cx + cy * cy) + cz * cz
        d = (qsq + csq) - 2.0 * (qx * cx + qy * cy + qz * cz)
        return d, lane + c * 128

    def _insert(vs, ids, nv, ni):
        lt = [nv < vs[s] for s in range(NSLOT)]
        ovs = [jnp.where(lt[0], nv, vs[0])]
        ois = [jnp.where(lt[0], ni, ids[0])]
        for s in range(1, NSLOT):
            ovs.append(jnp.where(lt[s],
                                 jnp.where(lt[s - 1], vs[s - 1], nv), vs[s]))
            ois.append(jnp.where(lt[s],
                                 jnp.where(lt[s - 1], ids[s - 1], ni), ids[s]))
        return tuple(ovs), tuple(ois)

    vs, ids = tuple([bigv] * NSLOT), tuple([bigi] * NSLOT)
    for c in range(CH):
        nv0, ni0 = _chunk_d_static(c)
        vs, ids = _insert(vs, ids, nv0, ni0)

    outb = jnp.zeros((RG, KNN), I32)
    lastv = jnp.full((RG, 1), -BIGF, F32)
    lasti = jnp.full((RG, 1), -1, I32)
    for t in range(KNN):
        cv, ci = bigv, bigi
        for s in range(NSLOT):
            ok = _lex_lt(lastv, lasti, vs[s], ids[s])
            sv = jnp.where(ok, vs[s], BIGF)
            si = jnp.where(ok, ids[s], BIGI)
            t2 = _lex_lt(sv, si, cv, ci)
            cv = jnp.where(t2, sv, cv)
            ci = jnp.where(t2, si, ci)
        mv = jnp.min(cv, axis=1, keepdims=True)
        mi = jnp.min(jnp.where(cv == mv, ci, BIGI), axis=1, keepdims=True)
        outb = jnp.where(outcol == t, jnp.broadcast_to(mi, (RG, KNN)), outb)
        lastv, lasti = mv, mi

    init = (jnp.zeros((RG, KNN), I32),
            jnp.full((RG, 1), -BIGF, F32), jnp.full((RG, 1), -1, I32))

    s7v, s7i = vs[NSLOT - 1], ids[NSLOT - 1]
    bad = jnp.any((s7v < lastv) | ((s7v == lastv) & (s7i <= lasti)))

    def slow(_):
        def ext1(t, st):
            outb, lastv, lasti = st

            def scan(c, st2):
                cv, ci = st2
                d, gi = _chunk_d(c)
                ok = _lex_lt(lastv, lasti, d, gi)
                sv = jnp.where(ok, d, BIGF)
                si = jnp.where(ok, gi, BIGI)
                t2 = _lex_lt(sv, si, cv, ci)
                return jnp.where(t2, sv, cv), jnp.where(t2, si, ci)

            cv, ci = lax.fori_loop(0, CH, scan, (bigv, bigi))
            mv = jnp.min(cv, axis=1, keepdims=True)
            mi = jnp.min(jnp.where(cv == mv, ci, BIGI), axis=1, keepdims=True)
            outb = jnp.where(outcol == t, jnp.broadcast_to(mi, (RG, KNN)), outb)
            return outb, mv, mi

        outb, _, _ = lax.fori_loop(0, KNN, ext1, init)
        return outb

    outb = lax.cond(bad, slow, lambda _: outb, 0)
    row = i * RG + lax.broadcasted_iota(I32, (RG, 1), 0)
    out_ref[...] = jnp.where(row < N_REAL, outb, SENT)


def _knn_topk(cand3, pts_pad):
    return pl.pallas_call(
        _knn_body,
        grid=(NP // RG,),
        in_specs=[
            pl.BlockSpec((3, CH, 128), lambda i: (0, 0, 0)),
            pl.BlockSpec((RG, 3), lambda i: (i, 0)),
        ],
        out_specs=pl.BlockSpec((RG, KNN), lambda i: (i, 0)),
        out_shape=jax.ShapeDtypeStruct((NP, KNN), I32),
    )(cand3, pts_pad)


@functools.cache
def _sc_hist_fn():
    @functools.partial(
        pl.kernel,
        mesh=_mesh(),
        out_type=jax.ShapeDtypeStruct((NW, NP), F32),
        scratch_types=[
            pltpu.VMEM((EPW,), I32),
            pltpu.VMEM((NP,), F32),
        ],
        compiler_params=pltpu.CompilerParams(needs_layout_passes=False),
    )
    def _sc_hist(idx_hbm, out_hbm, idx_v, hist_v):
        wid = lax.axis_index("s") * NC + lax.axis_index("c")
        pltpu.sync_copy(idx_hbm.at[wid], idx_v)
        zero16 = jnp.zeros((NL,), F32)
        one16 = jnp.full((NL,), 1.0, F32)

        def zbody(j, _):
            hist_v[pl.ds(j * NL, NL)] = zero16
            return 0

        lax.fori_loop(0, NP // NL, zbody, 0)

        def abody(j, _):
            v = idx_v[pl.ds(j * NL, NL)]
            plsc.addupdate_scatter(hist_v, [v], one16)
            return 0

        lax.fori_loop(0, EPW // NL, abody, 0)
        pltpu.sync_copy(hist_v, out_hbm.at[wid])

    return _sc_hist


def _deg_body(h_ref, o_ref):
    s = jnp.sum(h_ref[...], axis=0, keepdims=True)
    o_ref[...] = lax.rsqrt(jnp.maximum(s, 1.0))


def _deg_w(hist):
    return pl.pallas_call(
        _deg_body,
        out_shape=jax.ShapeDtypeStruct((1, NP), F32),
    )(hist)


def _scale_body(x_ref, w_ref, o_ref):
    o_ref[...] = x_ref[...] * w_ref[...]


def _scale_rows(x_pad, wcol):
    return pl.pallas_call(
        _scale_body,
        grid=(NP // RB,),
        in_specs=[
            pl.BlockSpec((RB, 128), lambda i: (i, 0)),
            pl.BlockSpec((RB, 1), lambda i: (i, 0)),
        ],
        out_specs=pl.BlockSpec((RB, 128), lambda i: (i, 0)),
        out_shape=jax.ShapeDtypeStruct((NP, 128), F32),
    )(x_pad, wcol)


def _tree_sum(vs):
    while len(vs) > 1:
        nxt = [vs[i] + vs[i + 1] for i in range(0, len(vs) - 1, 2)]
        if len(vs) % 2:
            nxt.append(vs[-1])
        vs = nxt
    return vs[0]


@functools.cache
def _sc_gather_fn():
    @functools.partial(
        pl.kernel,
        mesh=_mesh(),
        out_type=jax.ShapeDtypeStruct((NP, 128), F32),
        scratch_types=[
            pltpu.VMEM((NB, BN * KNN), I32),
            pltpu.VMEM((BN * KNN, 128), F32),
            pltpu.VMEM((BN, 128), F32),
            pltpu.SemaphoreType.DMA,
        ],
        compiler_params=pltpu.CompilerParams(needs_layout_passes=False),
    )
    def _sc_gather(hs_hbm, idx_hbm, out_hbm, idx_v, rows_v, out_v, sem):
        wid = lax.axis_index("s") * NC + lax.axis_index("c")
        pltpu.sync_copy(idx_hbm.at[wid], idx_v)

        def body(b, _):
            pltpu.async_copy(hs_hbm.at[idx_v.at[b]], rows_v, sem).wait()
            for n in range(BN):
                for c in range(8):
                    sl = pl.ds(c * NL, NL)
                    acc = _tree_sum([rows_v[n * KNN + m, sl] for m in range(KNN)])
                    out_v[n, sl] = acc
            pltpu.sync_copy(out_v, out_hbm.at[pl.ds(wid * NPW + b * BN, BN)])
            return 0

        lax.fori_loop(0, NB, body, 0)

    return _sc_gather


def _mm_body(a_ref, w_ref, b_ref, wc_ref, o_ref):
    a = a_ref[...] * INV_SQRT_K
    m = lax.dot_general(a, w_ref[...], (((1,), (0,)), ((), ())),
                        precision=lax.Precision.HIGHEST,
                        preferred_element_type=F32)
    h = jnp.maximum(m + b_ref[...], 0.0)
    o_ref[...] = h * wc_ref[...]


def _mm_layer(agg, W, b, wcol):
    return pl.pallas_call(
        _mm_body,
        grid=(NP // RB,),
        in_specs=[
            pl.BlockSpec((RB, 128), lambda i: (i, 0)),
            pl.BlockSpec((128, 128), lambda i: (0, 0)),
            pl.BlockSpec((1, 128), lambda i: (0, 0)),
            pl.BlockSpec((RB, 1), lambda i: (i, 0)),
        ],
        out_specs=pl.BlockSpec((RB, 128), lambda i: (i, 0)),
        out_shape=jax.ShapeDtypeStruct((NP, 128), F32),
    )(agg, W, b, wcol)


def _mm_final_body(a_ref, w_ref, b_ref, wf_ref, bf_ref, o_ref):
    a = a_ref[...] * INV_SQRT_K
    m = lax.dot_general(a, w_ref[...], (((1,), (0,)), ((), ())),
                        precision=lax.Precision.HIGHEST,
                        preferred_element_type=F32)
    h = jnp.maximum(m + b_ref[...], 0.0)
    z = lax.dot_general(h, wf_ref[...], (((1,), (0,)), ((), ())),
                        precision=lax.Precision.HIGHEST,
                        preferred_element_type=F32) + bf_ref[...]
    o_ref[...] = 1.0 / (1.0 + jnp.exp(-z))


def _mm_final(agg, W, b, Wf, bf):
    return pl.pallas_call(
        _mm_final_body,
        grid=(NP // RB,),
        in_specs=[
            pl.BlockSpec((RB, 128), lambda i: (i, 0)),
            pl.BlockSpec((128, 128), lambda i: (0, 0)),
            pl.BlockSpec((1, 128), lambda i: (0, 0)),
            pl.BlockSpec((128, 1), lambda i: (0, 0)),
            pl.BlockSpec((1, 1), lambda i: (0, 0)),
        ],
        out_specs=pl.BlockSpec((RB, 1), lambda i: (i, 0)),
        out_shape=jax.ShapeDtypeStruct((NP, 1), F32),
    )(agg, W, b, Wf, bf)


def kernel(x, triangle_centers, W1, b1, W2, b2, W3, b3, Wf, bf):
    pts = triangle_centers
    pad_n = NP - N_REAL
    pts_pad = jnp.pad(pts, ((0, pad_n), (0, 0)), constant_values=PADC)
    cand3 = jnp.pad(pts.T, ((0, 0), (0, pad_n)),
                    constant_values=PADC).reshape(3, CH, 128)
    x_pad = jnp.pad(x, ((0, pad_n), (0, 0)))

    idx = _knn_topk(cand3, pts_pad)
    hist = _sc_hist_fn()(idx.reshape(NW, EPW))
    w1d = _deg_w(hist)
    wcol = w1d.reshape(NP, 1)

    idx3 = idx.reshape(NW, NB, BN * KNN)
    h = _scale_rows(x_pad, wcol)
    for W, b in ((W1, b1), (W2, b2)):
        agg = _sc_gather_fn()(h, idx3)
        h = _mm_layer(agg, W, b.reshape(1, 128), wcol)
    agg = _sc_gather_fn()(h, idx3)
    res = _mm_final(agg, W3, b3.reshape(1, 128), Wf, bf.reshape(1, 1))
    return res[:N_REAL, 0]


# speedup vs baseline: 10.0305x; 1.3263x over previous
import functools

import jax
import jax.numpy as jnp
from jax import lax
from jax.experimental import pallas as pl
from jax.experimental.pallas import tpu as pltpu
from jax.experimental.pallas import tpu_sc as plsc

KNN = 32
N_REAL = 10000
NW = 32
NC, NS, NL = 2, 16, 16
NP = 10240
SENT = 10000
NPW = NP // NW
EPW = NPW * KNN
BN = 4
NB = NPW // BN
QB = 256
RB = 512
PADC = 1.0e18
BIGF = 3.0e38
INV_SQRT_K = float(1.0 / (32.0 ** 0.5))
F32 = jnp.float32
I32 = jnp.int32

@functools.cache
def _mesh():
    return plsc.VectorSubcoreMesh(core_axis_name="c", subcore_axis_name="s",
                                  num_cores=NC, num_subcores=NS)


CH = NP // 128
NSLOT = 6
RG = 64
BIGI = 2**30


def _lex_lt(av, ai, bv, bi):
    return (av < bv) | ((av == bv) & (ai < bi))


def _knn_body(cand3_ref, pts_ref, out_ref):
    i = pl.program_id(0)
    qx = pts_ref[:, 0:1]
    qy = pts_ref[:, 1:2]
    qz = pts_ref[:, 2:3]
    qsq = (qx * qx + qy * qy) + qz * qz
    lane = lax.broadcasted_iota(I32, (RG, 128), 1)
    bigv = jnp.full((RG, 128), BIGF, F32)
    bigi = jnp.full((RG, 128), BIGI, I32)
    outcol = lax.broadcasted_iota(I32, (RG, KNN), 1)

    def _chunk_d(c):
        cx = cand3_ref[0, pl.ds(c, 1), :]
        cy = cand3_ref[1, pl.ds(c, 1), :]
        cz = cand3_ref[2, pl.ds(c, 1), :]
        csq = (cx * cx + cy * cy) + cz * cz
        d = (qsq + csq) - 2.0 * (qx * cx + qy * cy + qz * cz)
        return d, lane + c * 128

    def _chunk_d_static(c):
        cx = cand3_ref[0, c, :].reshape(1, 128)
        cy = cand3_ref[1, c, :].reshape(1, 128)
        cz = cand3_ref[2, c, :].reshape(1, 128)
        csq = (cx * cx + cy * cy) + cz * cz
        d = (qsq + csq) - 2.0 * (qx * cx + qy * cy + qz * cz)
        return d, lane + c * 128

    def _insert(vs, ids, nv, ni):
        lt = [nv < vs[s] for s in range(NSLOT)]
        ovs = [jnp.where(lt[0], nv, vs[0])]
        ois = [jnp.where(lt[0], ni, ids[0])]
        for s in range(1, NSLOT):
            ovs.append(jnp.where(lt[s],
                                 jnp.where(lt[s - 1], vs[s - 1], nv), vs[s]))
            ois.append(jnp.where(lt[s],
                                 jnp.where(lt[s - 1], ids[s - 1], ni), ids[s]))
        return tuple(ovs), tuple(ois)

    vs, ids = tuple([bigv] * NSLOT), tuple([bigi] * NSLOT)
    for c in range(CH):
        nv0, ni0 = _chunk_d_static(c)
        vs, ids = _insert(vs, ids, nv0, ni0)

    outb = jnp.zeros((RG, KNN), I32)
    lastv = jnp.full((RG, 1), -BIGF, F32)
    lasti = jnp.full((RG, 1), -1, I32)
    for t in range(KNN):
        cv, ci = bigv, bigi
        for s in range(NSLOT):
            ok = _lex_lt(lastv, lasti, vs[s], ids[s])
            sv = jnp.where(ok, vs[s], BIGF)
            si = jnp.where(ok, ids[s], BIGI)
            t2 = _lex_lt(sv, si, cv, ci)
            cv = jnp.where(t2, sv, cv)
            ci = jnp.where(t2, si, ci)
        mv = jnp.min(cv, axis=1, keepdims=True)
        mi = jnp.min(jnp.where(cv == mv, ci, BIGI), axis=1, keepdims=True)
        outb = jnp.where(outcol == t, jnp.broadcast_to(mi, (RG, KNN)), outb)
        lastv, lasti = mv, mi

    init = (jnp.zeros((RG, KNN), I32),
            jnp.full((RG, 1), -BIGF, F32), jnp.full((RG, 1), -1, I32))

    s7v, s7i = vs[NSLOT - 1], ids[NSLOT - 1]
    bad = jnp.any((s7v < lastv) | ((s7v == lastv) & (s7i <= lasti)))

    def slow(_):
        def ext1(t, st):
            outb, lastv, lasti = st

            def scan(c, st2):
                cv, ci = st2
                d, gi = _chunk_d(c)
                ok = _lex_lt(lastv, lasti, d, gi)
                sv = jnp.where(ok, d, BIGF)
                si = jnp.where(ok, gi, BIGI)
                t2 = _lex_lt(sv, si, cv, ci)
                return jnp.where(t2, sv, cv), jnp.where(t2, si, ci)

            cv, ci = lax.fori_loop(0, CH, scan, (bigv, bigi))
            mv = jnp.min(cv, axis=1, keepdims=True)
            mi = jnp.min(jnp.where(cv == mv, ci, BIGI), axis=1, keepdims=True)
            outb = jnp.where(outcol == t, jnp.broadcast_to(mi, (RG, KNN)), outb)
            return outb, mv, mi

        outb, _, _ = lax.fori_loop(0, KNN, ext1, init)
        return outb

    outb = lax.cond(bad, slow, lambda _: outb, 0)
    row = i * RG + lax.broadcasted_iota(I32, (RG, 1), 0)
    out_ref[...] = jnp.where(row < N_REAL, outb, SENT)


def _knn_topk(cand3, pts_pad):
    return pl.pallas_call(
        _knn_body,
        grid=(NP // RG,),
        in_specs=[
            pl.BlockSpec((3, CH, 128), lambda i: (0, 0, 0)),
            pl.BlockSpec((RG, 3), lambda i: (i, 0)),
        ],
        out_specs=pl.BlockSpec((RG, KNN), lambda i: (i, 0)),
        out_shape=jax.ShapeDtypeStruct((NP, KNN), I32),
    )(cand3, pts_pad)


@functools.cache
def _sc_hist_fn():
    @functools.partial(
        pl.kernel,
        mesh=_mesh(),
        out_type=jax.ShapeDtypeStruct((NW, NP), F32),
        scratch_types=[
            pltpu.VMEM((EPW,), I32),
            pltpu.VMEM((NP,), F32),
        ],
        compiler_params=pltpu.CompilerParams(needs_layout_passes=False),
    )
    def _sc_hist(idx_hbm, out_hbm, idx_v, hist_v):
        wid = lax.axis_index("s") * NC + lax.axis_index("c")
        pltpu.sync_copy(idx_hbm.at[wid], idx_v)
        zero16 = jnp.zeros((NL,), F32)
        one16 = jnp.full((NL,), 1.0, F32)

        def zbody(j, _):
            hist_v[pl.ds(j * NL, NL)] = zero16
            return 0

        lax.fori_loop(0, NP // NL, zbody, 0)

        def abody(j, _):
            v = idx_v[pl.ds(j * NL, NL)]
            plsc.addupdate_scatter(hist_v, [v], one16)
            return 0

        lax.fori_loop(0, EPW // NL, abody, 0)
        pltpu.sync_copy(hist_v, out_hbm.at[wid])

    return _sc_hist


def _deg_body(h_ref, o_ref):
    s = jnp.sum(h_ref[...], axis=0, keepdims=True)
    o_ref[...] = lax.rsqrt(jnp.maximum(s, 1.0))


def _deg_w(hist):
    return pl.pallas_call(
        _deg_body,
        out_shape=jax.ShapeDtypeStruct((1, NP), F32),
    )(hist)


def _scale_body(x_ref, w_ref, o_ref):
    o_ref[...] = x_ref[...] * w_ref[...]


def _scale_rows(x_pad, wcol):
    return pl.pallas_call(
        _scale_body,
        grid=(NP // RB,),
        in_specs=[
            pl.BlockSpec((RB, 128), lambda i: (i, 0)),
            pl.BlockSpec((RB, 1), lambda i: (i, 0)),
        ],
        out_specs=pl.BlockSpec((RB, 128), lambda i: (i, 0)),
        out_shape=jax.ShapeDtypeStruct((NP, 128), F32),
    )(x_pad, wcol)


def _tree_sum(vs):
    while len(vs) > 1:
        nxt = [vs[i] + vs[i + 1] for i in range(0, len(vs) - 1, 2)]
        if len(vs) % 2:
            nxt.append(vs[-1])
        vs = nxt
    return vs[0]


@functools.cache
def _sc_gather_fn():
    @functools.partial(
        pl.kernel,
        mesh=_mesh(),
        out_type=jax.ShapeDtypeStruct((NP, 128), F32),
        scratch_types=[
            pltpu.VMEM((NB, BN * KNN), I32),
            pltpu.VMEM((BN * KNN, 128), F32),
            pltpu.VMEM((BN, 128), F32),
            pltpu.SemaphoreType.DMA,
        ],
        compiler_params=pltpu.CompilerParams(needs_layout_passes=False),
    )
    def _sc_gather(hs_hbm, idx_hbm, out_hbm, idx_v, rows_v, out_v, sem):
        wid = lax.axis_index("s") * NC + lax.axis_index("c")
        pltpu.sync_copy(idx_hbm.at[wid], idx_v)

        def body(b, _):
            pltpu.async_copy(hs_hbm.at[idx_v.at[b]], rows_v, sem).wait()
            for n in range(BN):
                for c in range(8):
                    sl = pl.ds(c * NL, NL)
                    acc = _tree_sum([rows_v[n * KNN + m, sl] for m in range(KNN)])
                    out_v[n, sl] = acc
            pltpu.sync_copy(out_v, out_hbm.at[pl.ds(wid * NPW + b * BN, BN)])
            return 0

        lax.fori_loop(0, NB, body, 0)

    return _sc_gather


def _mm_body(a_ref, w_ref, b_ref, wc_ref, o_ref):
    a = a_ref[...] * INV_SQRT_K
    m = lax.dot_general(a, w_ref[...], (((1,), (0,)), ((), ())),
                        precision=lax.Precision.HIGHEST,
                        preferred_element_type=F32)
    h = jnp.maximum(m + b_ref[...], 0.0)
    o_ref[...] = h * wc_ref[...]


def _mm_layer(agg, W, b, wcol):
    return pl.pallas_call(
        _mm_body,
        grid=(NP // RB,),
        in_specs=[
            pl.BlockSpec((RB, 128), lambda i: (i, 0)),
            pl.BlockSpec((128, 128), lambda i: (0, 0)),
            pl.BlockSpec((1, 128), lambda i: (0, 0)),
            pl.BlockSpec((RB, 1), lambda i: (i, 0)),
        ],
        out_specs=pl.BlockSpec((RB, 128), lambda i: (i, 0)),
        out_shape=jax.ShapeDtypeStruct((NP, 128), F32),
    )(agg, W, b, wcol)


def _mm_final_body(a_ref, w_ref, b_ref, wf_ref, bf_ref, o_ref):
    a = a_ref[...] * INV_SQRT_K
    m = lax.dot_general(a, w_ref[...], (((1,), (0,)), ((), ())),
                        precision=lax.Precision.HIGHEST,
                        preferred_element_type=F32)
    h = jnp.maximum(m + b_ref[...], 0.0)
    z = lax.dot_general(h, wf_ref[...], (((1,), (0,)), ((), ())),
                        precision=lax.Precision.HIGHEST,
                        preferred_element_type=F32) + bf_ref[...]
    o_ref[...] = 1.0 / (1.0 + jnp.exp(-z))


def _mm_final(agg, W, b, Wf, bf):
    return pl.pallas_call(
        _mm_final_body,
        grid=(NP // RB,),
        in_specs=[
            pl.BlockSpec((RB, 128), lambda i: (i, 0)),
            pl.BlockSpec((128, 128), lambda i: (0, 0)),
            pl.BlockSpec((1, 128), lambda i: (0, 0)),
            pl.BlockSpec((128, 1), lambda i: (0, 0)),
            pl.BlockSpec((1, 1), lambda i: (0, 0)),
        ],
        out_specs=pl.BlockSpec((RB, 1), lambda i: (i, 0)),
        out_shape=jax.ShapeDtypeStruct((NP, 1), F32),
    )(agg, W, b, Wf, bf)


def kernel(x, triangle_centers, W1, b1, W2, b2, W3, b3, Wf, bf):
    pts = triangle_centers
    pad_n = NP - N_REAL
    pts_pad = jnp.pad(pts, ((0, pad_n), (0, 0)), constant_values=PADC)
    cand3 = jnp.pad(pts.T, ((0, 0), (0, pad_n)),
                    constant_values=PADC).reshape(3, CH, 128)
    x_pad = jnp.pad(x, ((0, pad_n), (0, 0)))

    idx = _knn_topk(cand3, pts_pad)
    hist = _sc_hist_fn()(idx.reshape(NW, EPW))
    w1d = _deg_w(hist)
    wcol = w1d.reshape(NP, 1)

    idx3 = idx.reshape(NW, NB, BN * KNN)
    h = _scale_rows(x_pad, wcol)
    for W, b in ((W1, b1), (W2, b2)):
        agg = _sc_gather_fn()(h, idx3)
        h = _mm_layer(agg, W, b.reshape(1, 128), wcol)
    agg = _sc_gather_fn()(h, idx3)
    res = _mm_final(agg, W3, b3.reshape(1, 128), Wf, bf.reshape(1, 1))
    return res[:N_REAL, 0]


# speedup vs baseline: 11.4717x; 1.1437x over previous
import functools

import jax
import jax.numpy as jnp
from jax import lax
from jax.experimental import pallas as pl
from jax.experimental.pallas import tpu as pltpu
from jax.experimental.pallas import tpu_sc as plsc

KNN = 32
N_REAL = 10000
NW = 32
NC, NS, NL = 2, 16, 16
NP = 10240
SENT = 10000
NPW = NP // NW
EPW = NPW * KNN
BN = 4
NB = NPW // BN
QB = 256
RB = 512
PADC = 1.0e18
BIGF = 3.0e38
INV_SQRT_K = float(1.0 / (32.0 ** 0.5))
F32 = jnp.float32
I32 = jnp.int32

@functools.cache
def _mesh():
    return plsc.VectorSubcoreMesh(core_axis_name="c", subcore_axis_name="s",
                                  num_cores=NC, num_subcores=NS)


CH = NP // 128
NSLOT = 6
RG = 128
BIGI = 2**30


def _lex_lt(av, ai, bv, bi):
    return (av < bv) | ((av == bv) & (ai < bi))


def _knn_body(cand3_ref, pts_ref, out_ref):
    i = pl.program_id(0)
    qx = pts_ref[:, 0:1]
    qy = pts_ref[:, 1:2]
    qz = pts_ref[:, 2:3]
    qsq = (qx * qx + qy * qy) + qz * qz
    lane = lax.broadcasted_iota(I32, (RG, 128), 1)
    bigv = jnp.full((RG, 128), BIGF, F32)
    bigi = jnp.full((RG, 128), BIGI, I32)
    outcol = lax.broadcasted_iota(I32, (RG, KNN), 1)

    def _chunk_d(c):
        cx = cand3_ref[0, pl.ds(c, 1), :]
        cy = cand3_ref[1, pl.ds(c, 1), :]
        cz = cand3_ref[2, pl.ds(c, 1), :]
        csq = (cx * cx + cy * cy) + cz * cz
        d = (qsq + csq) - 2.0 * (qx * cx + qy * cy + qz * cz)
        return d, lane + c * 128

    def _chunk_d_static(c):
        cx = cand3_ref[0, c, :].reshape(1, 128)
        cy = cand3_ref[1, c, :].reshape(1, 128)
        cz = cand3_ref[2, c, :].reshape(1, 128)
        csq = (cx * cx + cy * cy) + cz * cz
        d = (qsq + csq) - 2.0 * (qx * cx + qy * cy + qz * cz)
        return d, lane + c * 128

    def _insert(vs, ids, nv, ni):
        lt = [nv < vs[s] for s in range(NSLOT)]
        ovs = [jnp.where(lt[0], nv, vs[0])]
        ois = [jnp.where(lt[0], ni, ids[0])]
        for s in range(1, NSLOT):
            ovs.append(jnp.where(lt[s],
                                 jnp.where(lt[s - 1], vs[s - 1], nv), vs[s]))
            ois.append(jnp.where(lt[s],
                                 jnp.where(lt[s - 1], ids[s - 1], ni), ids[s]))
        return tuple(ovs), tuple(ois)

    vs, ids = tuple([bigv] * NSLOT), tuple([bigi] * NSLOT)
    for c in range(CH):
        nv0, ni0 = _chunk_d_static(c)
        vs, ids = _insert(vs, ids, nv0, ni0)

    outb = jnp.zeros((RG, KNN), I32)
    lastv = jnp.full((RG, 1), -BIGF, F32)
    lasti = jnp.full((RG, 1), -1, I32)
    for t in range(KNN):
        cv, ci = bigv, bigi
        for s in range(NSLOT):
            ok = _lex_lt(lastv, lasti, vs[s], ids[s])
            sv = jnp.where(ok, vs[s], BIGF)
            si = jnp.where(ok, ids[s], BIGI)
            t2 = _lex_lt(sv, si, cv, ci)
            cv = jnp.where(t2, sv, cv)
            ci = jnp.where(t2, si, ci)
        mv = jnp.min(cv, axis=1, keepdims=True)
        mi = jnp.min(jnp.where(cv == mv, ci, BIGI), axis=1, keepdims=True)
        outb = jnp.where(outcol == t, jnp.broadcast_to(mi, (RG, KNN)), outb)
        lastv, lasti = mv, mi

    init = (jnp.zeros((RG, KNN), I32),
            jnp.full((RG, 1), -BIGF, F32), jnp.full((RG, 1), -1, I32))

    s7v, s7i = vs[NSLOT - 1], ids[NSLOT - 1]
    bad = jnp.any((s7v < lastv) | ((s7v == lastv) & (s7i <= lasti)))

    def slow(_):
        def ext1(t, st):
            outb, lastv, lasti = st

            def scan(c, st2):
                cv, ci = st2
                d, gi = _chunk_d(c)
                ok = _lex_lt(lastv, lasti, d, gi)
                sv = jnp.where(ok, d, BIGF)
                si = jnp.where(ok, gi, BIGI)
                t2 = _lex_lt(sv, si, cv, ci)
                return jnp.where(t2, sv, cv), jnp.where(t2, si, ci)

            cv, ci = lax.fori_loop(0, CH, scan, (bigv, bigi))
            mv = jnp.min(cv, axis=1, keepdims=True)
            mi = jnp.min(jnp.where(cv == mv, ci, BIGI), axis=1, keepdims=True)
            outb = jnp.where(outcol == t, jnp.broadcast_to(mi, (RG, KNN)), outb)
            return outb, mv, mi

        outb, _, _ = lax.fori_loop(0, KNN, ext1, init)
        return outb

    outb = lax.cond(bad, slow, lambda _: outb, 0)
    row = i * RG + lax.broadcasted_iota(I32, (RG, 1), 0)
    out_ref[...] = jnp.where(row < N_REAL, outb, SENT)


def _knn_topk(cand3, pts_pad):
    return pl.pallas_call(
        _knn_body,
        grid=(NP // RG,),
        in_specs=[
            pl.BlockSpec((3, CH, 128), lambda i: (0, 0, 0)),
            pl.BlockSpec((RG, 3), lambda i: (i, 0)),
        ],
        out_specs=pl.BlockSpec((RG, KNN), lambda i: (i, 0)),
        out_shape=jax.ShapeDtypeStruct((NP, KNN), I32),
    )(cand3, pts_pad)


@functools.cache
def _sc_hist_fn():
    @functools.partial(
        pl.kernel,
        mesh=_mesh(),
        out_type=jax.ShapeDtypeStruct((NW, NP), F32),
        scratch_types=[
            pltpu.VMEM((EPW,), I32),
            pltpu.VMEM((NP,), F32),
        ],
        compiler_params=pltpu.CompilerParams(needs_layout_passes=False),
    )
    def _sc_hist(idx_hbm, out_hbm, idx_v, hist_v):
        wid = lax.axis_index("s") * NC + lax.axis_index("c")
        pltpu.sync_copy(idx_hbm.at[wid], idx_v)
        zero16 = jnp.zeros((NL,), F32)
        one16 = jnp.full((NL,), 1.0, F32)

        def zbody(j, _):
            hist_v[pl.ds(j * NL, NL)] = zero16
            return 0

        lax.fori_loop(0, NP // NL, zbody, 0)

        def abody(j, _):
            v = idx_v[pl.ds(j * NL, NL)]
            plsc.addupdate_scatter(hist_v, [v], one16)
            return 0

        lax.fori_loop(0, EPW // NL, abody, 0)
        pltpu.sync_copy(hist_v, out_hbm.at[wid])

    return _sc_hist


def _deg_body(h_ref, o_ref):
    s = jnp.sum(h_ref[...], axis=0, keepdims=True)
    o_ref[...] = lax.rsqrt(jnp.maximum(s, 1.0))


def _deg_w(hist):
    return pl.pallas_call(
        _deg_body,
        out_shape=jax.ShapeDtypeStruct((1, NP), F32),
    )(hist)


def _scale_body(x_ref, w_ref, o_ref):
    o_ref[...] = x_ref[...] * w_ref[...]


def _scale_rows(x_pad, wcol):
    return pl.pallas_call(
        _scale_body,
        grid=(NP // RB,),
        in_specs=[
            pl.BlockSpec((RB, 128), lambda i: (i, 0)),
            pl.BlockSpec((RB, 1), lambda i: (i, 0)),
        ],
        out_specs=pl.BlockSpec((RB, 128), lambda i: (i, 0)),
        out_shape=jax.ShapeDtypeStruct((NP, 128), F32),
    )(x_pad, wcol)


def _tree_sum(vs):
    while len(vs) > 1:
        nxt = [vs[i] + vs[i + 1] for i in range(0, len(vs) - 1, 2)]
        if len(vs) % 2:
            nxt.append(vs[-1])
        vs = nxt
    return vs[0]


@functools.cache
def _sc_gather_fn():
    @functools.partial(
        pl.kernel,
        mesh=_mesh(),
        out_type=jax.ShapeDtypeStruct((NP, 128), F32),
        scratch_types=[
            pltpu.VMEM((NB, BN * KNN), I32),
            pltpu.VMEM((BN * KNN, 128), F32),
            pltpu.VMEM((BN, 128), F32),
            pltpu.SemaphoreType.DMA,
        ],
        compiler_params=pltpu.CompilerParams(needs_layout_passes=False),
    )
    def _sc_gather(hs_hbm, idx_hbm, out_hbm, idx_v, rows_v, out_v, sem):
        wid = lax.axis_index("s") * NC + lax.axis_index("c")
        pltpu.sync_copy(idx_hbm.at[wid], idx_v)

        def body(b, _):
            pltpu.async_copy(hs_hbm.at[idx_v.at[b]], rows_v, sem).wait()
            for n in range(BN):
                for c in range(8):
                    sl = pl.ds(c * NL, NL)
                    acc = _tree_sum([rows_v[n * KNN + m, sl] for m in range(KNN)])
                    out_v[n, sl] = acc
            pltpu.sync_copy(out_v, out_hbm.at[pl.ds(wid * NPW + b * BN, BN)])
            return 0

        lax.fori_loop(0, NB, body, 0)

    return _sc_gather


def _mm_body(a_ref, w_ref, b_ref, wc_ref, o_ref):
    a = a_ref[...] * INV_SQRT_K
    m = lax.dot_general(a, w_ref[...], (((1,), (0,)), ((), ())),
                        precision=lax.Precision.HIGHEST,
                        preferred_element_type=F32)
    h = jnp.maximum(m + b_ref[...], 0.0)
    o_ref[...] = h * wc_ref[...]


def _mm_layer(agg, W, b, wcol):
    return pl.pallas_call(
        _mm_body,
        grid=(NP // RB,),
        in_specs=[
            pl.BlockSpec((RB, 128), lambda i: (i, 0)),
            pl.BlockSpec((128, 128), lambda i: (0, 0)),
            pl.BlockSpec((1, 128), lambda i: (0, 0)),
            pl.BlockSpec((RB, 1), lambda i: (i, 0)),
        ],
        out_specs=pl.BlockSpec((RB, 128), lambda i: (i, 0)),
        out_shape=jax.ShapeDtypeStruct((NP, 128), F32),
    )(agg, W, b, wcol)


def _mm_final_body(a_ref, w_ref, b_ref, wf_ref, bf_ref, o_ref):
    a = a_ref[...] * INV_SQRT_K
    m = lax.dot_general(a, w_ref[...], (((1,), (0,)), ((), ())),
                        precision=lax.Precision.HIGHEST,
                        preferred_element_type=F32)
    h = jnp.maximum(m + b_ref[...], 0.0)
    z = lax.dot_general(h, wf_ref[...], (((1,), (0,)), ((), ())),
                        precision=lax.Precision.HIGHEST,
                        preferred_element_type=F32) + bf_ref[...]
    o_ref[...] = 1.0 / (1.0 + jnp.exp(-z))


def _mm_final(agg, W, b, Wf, bf):
    return pl.pallas_call(
        _mm_final_body,
        grid=(NP // RB,),
        in_specs=[
            pl.BlockSpec((RB, 128), lambda i: (i, 0)),
            pl.BlockSpec((128, 128), lambda i: (0, 0)),
            pl.BlockSpec((1, 128), lambda i: (0, 0)),
            pl.BlockSpec((128, 1), lambda i: (0, 0)),
            pl.BlockSpec((1, 1), lambda i: (0, 0)),
        ],
        out_specs=pl.BlockSpec((RB, 1), lambda i: (i, 0)),
        out_shape=jax.ShapeDtypeStruct((NP, 1), F32),
    )(agg, W, b, Wf, bf)


def kernel(x, triangle_centers, W1, b1, W2, b2, W3, b3, Wf, bf):
    pts = triangle_centers
    pad_n = NP - N_REAL
    pts_pad = jnp.pad(pts, ((0, pad_n), (0, 0)), constant_values=PADC)
    cand3 = jnp.pad(pts.T, ((0, 0), (0, pad_n)),
                    constant_values=PADC).reshape(3, CH, 128)
    x_pad = jnp.pad(x, ((0, pad_n), (0, 0)))

    idx = _knn_topk(cand3, pts_pad)
    hist = _sc_hist_fn()(idx.reshape(NW, EPW))
    w1d = _deg_w(hist)
    wcol = w1d.reshape(NP, 1)

    idx3 = idx.reshape(NW, NB, BN * KNN)
    h = _scale_rows(x_pad, wcol)
    for W, b in ((W1, b1), (W2, b2)):
        agg = _sc_gather_fn()(h, idx3)
        h = _mm_layer(agg, W, b.reshape(1, 128), wcol)
    agg = _sc_gather_fn()(h, idx3)
    res = _mm_final(agg, W3, b3.reshape(1, 128), Wf, bf.reshape(1, 1))
    return res[:N_REAL, 0]


# speedup vs baseline: 12.1775x; 1.0615x over previous
import functools

import jax
import jax.numpy as jnp
from jax import lax
from jax.experimental import pallas as pl
from jax.experimental.pallas import tpu as pltpu
from jax.experimental.pallas import tpu_sc as plsc

KNN = 32
N_REAL = 10000
NW = 32
NC, NS, NL = 2, 16, 16
NP = 10240
SENT = 10000
NPW = NP // NW
EPW = NPW * KNN
BN = 4
NB = NPW // BN
QB = 256
RB = 512
PADC = 1.0e18
BIGF = 3.0e38
INV_SQRT_K = float(1.0 / (32.0 ** 0.5))
F32 = jnp.float32
I32 = jnp.int32

@functools.cache
def _mesh():
    return plsc.VectorSubcoreMesh(core_axis_name="c", subcore_axis_name="s",
                                  num_cores=NC, num_subcores=NS)


CH = NP // 128
NSLOT = 6
RG = 256
BIGI = 2**30


def _lex_lt(av, ai, bv, bi):
    return (av < bv) | ((av == bv) & (ai < bi))


def _knn_body(cand3_ref, pts_ref, out_ref):
    i = pl.program_id(0)
    qx = pts_ref[:, 0:1]
    qy = pts_ref[:, 1:2]
    qz = pts_ref[:, 2:3]
    qsq = (qx * qx + qy * qy) + qz * qz
    lane = lax.broadcasted_iota(I32, (RG, 128), 1)
    bigv = jnp.full((RG, 128), BIGF, F32)
    bigi = jnp.full((RG, 128), BIGI, I32)
    outcol = lax.broadcasted_iota(I32, (RG, KNN), 1)

    def _chunk_d(c):
        cx = cand3_ref[0, pl.ds(c, 1), :]
        cy = cand3_ref[1, pl.ds(c, 1), :]
        cz = cand3_ref[2, pl.ds(c, 1), :]
        csq = (cx * cx + cy * cy) + cz * cz
        d = (qsq + csq) - 2.0 * (qx * cx + qy * cy + qz * cz)
        return d, lane + c * 128

    def _chunk_d_static(c):
        cx = cand3_ref[0, c, :].reshape(1, 128)
        cy = cand3_ref[1, c, :].reshape(1, 128)
        cz = cand3_ref[2, c, :].reshape(1, 128)
        csq = (cx * cx + cy * cy) + cz * cz
        d = (qsq + csq) - 2.0 * (qx * cx + qy * cy + qz * cz)
        return d, lane + c * 128

    def _insert(vs, ids, nv, ni):
        lt = [nv < vs[s] for s in range(NSLOT)]
        ovs = [jnp.where(lt[0], nv, vs[0])]
        ois = [jnp.where(lt[0], ni, ids[0])]
        for s in range(1, NSLOT):
            ovs.append(jnp.where(lt[s],
                                 jnp.where(lt[s - 1], vs[s - 1], nv), vs[s]))
            ois.append(jnp.where(lt[s],
                                 jnp.where(lt[s - 1], ids[s - 1], ni), ids[s]))
        return tuple(ovs), tuple(ois)

    vs, ids = tuple([bigv] * NSLOT), tuple([bigi] * NSLOT)
    for c in range(CH):
        nv0, ni0 = _chunk_d_static(c)
        vs, ids = _insert(vs, ids, nv0, ni0)

    outb = jnp.zeros((RG, KNN), I32)
    lastv = jnp.full((RG, 1), -BIGF, F32)
    lasti = jnp.full((RG, 1), -1, I32)
    for t in range(KNN):
        cv, ci = bigv, bigi
        for s in range(NSLOT):
            ok = _lex_lt(lastv, lasti, vs[s], ids[s])
            sv = jnp.where(ok, vs[s], BIGF)
            si = jnp.where(ok, ids[s], BIGI)
            t2 = _lex_lt(sv, si, cv, ci)
            cv = jnp.where(t2, sv, cv)
            ci = jnp.where(t2, si, ci)
        mv = jnp.min(cv, axis=1, keepdims=True)
        mi = jnp.min(jnp.where(cv == mv, ci, BIGI), axis=1, keepdims=True)
        outb = jnp.where(outcol == t, jnp.broadcast_to(mi, (RG, KNN)), outb)
        lastv, lasti = mv, mi

    init = (jnp.zeros((RG, KNN), I32),
            jnp.full((RG, 1), -BIGF, F32), jnp.full((RG, 1), -1, I32))

    s7v, s7i = vs[NSLOT - 1], ids[NSLOT - 1]
    bad = jnp.any((s7v < lastv) | ((s7v == lastv) & (s7i <= lasti)))

    def slow(_):
        def ext1(t, st):
            outb, lastv, lasti = st

            def scan(c, st2):
                cv, ci = st2
                d, gi = _chunk_d(c)
                ok = _lex_lt(lastv, lasti, d, gi)
                sv = jnp.where(ok, d, BIGF)
                si = jnp.where(ok, gi, BIGI)
                t2 = _lex_lt(sv, si, cv, ci)
                return jnp.where(t2, sv, cv), jnp.where(t2, si, ci)

            cv, ci = lax.fori_loop(0, CH, scan, (bigv, bigi))
            mv = jnp.min(cv, axis=1, keepdims=True)
            mi = jnp.min(jnp.where(cv == mv, ci, BIGI), axis=1, keepdims=True)
            outb = jnp.where(outcol == t, jnp.broadcast_to(mi, (RG, KNN)), outb)
            return outb, mv, mi

        outb, _, _ = lax.fori_loop(0, KNN, ext1, init)
        return outb

    outb = lax.cond(bad, slow, lambda _: outb, 0)
    row = i * RG + lax.broadcasted_iota(I32, (RG, 1), 0)
    out_ref[...] = jnp.where(row < N_REAL, outb, SENT)


def _knn_topk(cand3, pts_pad):
    return pl.pallas_call(
        _knn_body,
        grid=(NP // RG,),
        in_specs=[
            pl.BlockSpec((3, CH, 128), lambda i: (0, 0, 0)),
            pl.BlockSpec((RG, 3), lambda i: (i, 0)),
        ],
        out_specs=pl.BlockSpec((RG, KNN), lambda i: (i, 0)),
        out_shape=jax.ShapeDtypeStruct((NP, KNN), I32),
    )(cand3, pts_pad)


@functools.cache
def _sc_hist_fn():
    @functools.partial(
        pl.kernel,
        mesh=_mesh(),
        out_type=jax.ShapeDtypeStruct((NW, NP), F32),
        scratch_types=[
            pltpu.VMEM((EPW,), I32),
            pltpu.VMEM((NP,), F32),
        ],
        compiler_params=pltpu.CompilerParams(needs_layout_passes=False),
    )
    def _sc_hist(idx_hbm, out_hbm, idx_v, hist_v):
        wid = lax.axis_index("s") * NC + lax.axis_index("c")
        pltpu.sync_copy(idx_hbm.at[wid], idx_v)
        zero16 = jnp.zeros((NL,), F32)
        one16 = jnp.full((NL,), 1.0, F32)

        def zbody(j, _):
            hist_v[pl.ds(j * NL, NL)] = zero16
            return 0

        lax.fori_loop(0, NP // NL, zbody, 0)

        def abody(j, _):
            v = idx_v[pl.ds(j * NL, NL)]
            plsc.addupdate_scatter(hist_v, [v], one16)
            return 0

        lax.fori_loop(0, EPW // NL, abody, 0)
        pltpu.sync_copy(hist_v, out_hbm.at[wid])

    return _sc_hist


def _deg_body(h_ref, o_ref):
    s = jnp.sum(h_ref[...], axis=0, keepdims=True)
    o_ref[...] = lax.rsqrt(jnp.maximum(s, 1.0))


def _deg_w(hist):
    return pl.pallas_call(
        _deg_body,
        out_shape=jax.ShapeDtypeStruct((1, NP), F32),
    )(hist)


def _scale_body(x_ref, w_ref, o_ref):
    o_ref[...] = x_ref[...] * w_ref[...]


def _scale_rows(x_pad, wcol):
    return pl.pallas_call(
        _scale_body,
        grid=(NP // RB,),
        in_specs=[
            pl.BlockSpec((RB, 128), lambda i: (i, 0)),
            pl.BlockSpec((RB, 1), lambda i: (i, 0)),
        ],
        out_specs=pl.BlockSpec((RB, 128), lambda i: (i, 0)),
        out_shape=jax.ShapeDtypeStruct((NP, 128), F32),
    )(x_pad, wcol)


def _tree_sum(vs):
    while len(vs) > 1:
        nxt = [vs[i] + vs[i + 1] for i in range(0, len(vs) - 1, 2)]
        if len(vs) % 2:
            nxt.append(vs[-1])
        vs = nxt
    return vs[0]


@functools.cache
def _sc_gather_fn():
    @functools.partial(
        pl.kernel,
        mesh=_mesh(),
        out_type=jax.ShapeDtypeStruct((NP, 128), F32),
        scratch_types=[
            pltpu.VMEM((NB, BN * KNN), I32),
            pltpu.VMEM((BN * KNN, 128), F32),
            pltpu.VMEM((BN, 128), F32),
            pltpu.SemaphoreType.DMA,
        ],
        compiler_params=pltpu.CompilerParams(needs_layout_passes=False),
    )
    def _sc_gather(hs_hbm, idx_hbm, out_hbm, idx_v, rows_v, out_v, sem):
        wid = lax.axis_index("s") * NC + lax.axis_index("c")
        pltpu.sync_copy(idx_hbm.at[wid], idx_v)

        def body(b, _):
            pltpu.async_copy(hs_hbm.at[idx_v.at[b]], rows_v, sem).wait()
            for n in range(BN):
                for c in range(8):
                    sl = pl.ds(c * NL, NL)
                    acc = _tree_sum([rows_v[n * KNN + m, sl] for m in range(KNN)])
                    out_v[n, sl] = acc
            pltpu.sync_copy(out_v, out_hbm.at[pl.ds(wid * NPW + b * BN, BN)])
            return 0

        lax.fori_loop(0, NB, body, 0)

    return _sc_gather


def _mm_body(a_ref, w_ref, b_ref, wc_ref, o_ref):
    a = a_ref[...] * INV_SQRT_K
    m = lax.dot_general(a, w_ref[...], (((1,), (0,)), ((), ())),
                        precision=lax.Precision.HIGHEST,
                        preferred_element_type=F32)
    h = jnp.maximum(m + b_ref[...], 0.0)
    o_ref[...] = h * wc_ref[...]


def _mm_layer(agg, W, b, wcol):
    return pl.pallas_call(
        _mm_body,
        grid=(NP // RB,),
        in_specs=[
            pl.BlockSpec((RB, 128), lambda i: (i, 0)),
            pl.BlockSpec((128, 128), lambda i: (0, 0)),
            pl.BlockSpec((1, 128), lambda i: (0, 0)),
            pl.BlockSpec((RB, 1), lambda i: (i, 0)),
        ],
        out_specs=pl.BlockSpec((RB, 128), lambda i: (i, 0)),
        out_shape=jax.ShapeDtypeStruct((NP, 128), F32),
    )(agg, W, b, wcol)


def _mm_final_body(a_ref, w_ref, b_ref, wf_ref, bf_ref, o_ref):
    a = a_ref[...] * INV_SQRT_K
    m = lax.dot_general(a, w_ref[...], (((1,), (0,)), ((), ())),
                        precision=lax.Precision.HIGHEST,
                        preferred_element_type=F32)
    h = jnp.maximum(m + b_ref[...], 0.0)
    z = lax.dot_general(h, wf_ref[...], (((1,), (0,)), ((), ())),
                        precision=lax.Precision.HIGHEST,
                        preferred_element_type=F32) + bf_ref[...]
    o_ref[...] = 1.0 / (1.0 + jnp.exp(-z))


def _mm_final(agg, W, b, Wf, bf):
    return pl.pallas_call(
        _mm_final_body,
        grid=(NP // RB,),
        in_specs=[
            pl.BlockSpec((RB, 128), lambda i: (i, 0)),
            pl.BlockSpec((128, 128), lambda i: (0, 0)),
            pl.BlockSpec((1, 128), lambda i: (0, 0)),
            pl.BlockSpec((128, 1), lambda i: (0, 0)),
            pl.BlockSpec((1, 1), lambda i: (0, 0)),
        ],
        out_specs=pl.BlockSpec((RB, 1), lambda i: (i, 0)),
        out_shape=jax.ShapeDtypeStruct((NP, 1), F32),
    )(agg, W, b, Wf, bf)


def kernel(x, triangle_centers, W1, b1, W2, b2, W3, b3, Wf, bf):
    pts = triangle_centers
    pad_n = NP - N_REAL
    pts_pad = jnp.pad(pts, ((0, pad_n), (0, 0)), constant_values=PADC)
    cand3 = jnp.pad(pts.T, ((0, 0), (0, pad_n)),
                    constant_values=PADC).reshape(3, CH, 128)
    x_pad = jnp.pad(x, ((0, pad_n), (0, 0)))

    idx = _knn_topk(cand3, pts_pad)
    hist = _sc_hist_fn()(idx.reshape(NW, EPW))
    w1d = _deg_w(hist)
    wcol = w1d.reshape(NP, 1)

    idx3 = idx.reshape(NW, NB, BN * KNN)
    h = _scale_rows(x_pad, wcol)
    for W, b in ((W1, b1), (W2, b2)):
        agg = _sc_gather_fn()(h, idx3)
        h = _mm_layer(agg, W, b.reshape(1, 128), wcol)
    agg = _sc_gather_fn()(h, idx3)
    res = _mm_final(agg, W3, b3.reshape(1, 128), Wf, bf.reshape(1, 1))
    return res[:N_REAL, 0]


# speedup vs baseline: 13.9714x; 1.1473x over previous
import functools

import jax
import jax.numpy as jnp
from jax import lax
from jax.experimental import pallas as pl
from jax.experimental.pallas import tpu as pltpu
from jax.experimental.pallas import tpu_sc as plsc

KNN = 32
N_REAL = 10000
NW = 32
NC, NS, NL = 2, 16, 16
NP = 10240
SENT = 10000
NPW = NP // NW
EPW = NPW * KNN
BN = 4
NB = NPW // BN
QB = 256
RB = 512
PADC = 1.0e18
BIGF = 3.0e38
INV_SQRT_K = float(1.0 / (32.0 ** 0.5))
F32 = jnp.float32
I32 = jnp.int32

@functools.cache
def _mesh():
    return plsc.VectorSubcoreMesh(core_axis_name="c", subcore_axis_name="s",
                                  num_cores=NC, num_subcores=NS)


CH = NP // 128
NSLOT = 6
RG = 256
BIGI = 2**30


def _lex_lt(av, ai, bv, bi):
    return (av < bv) | ((av == bv) & (ai < bi))


def _knn_body(cand3_ref, pts_ref, out_ref):
    i = pl.program_id(0)
    qx = pts_ref[:, 0:1]
    qy = pts_ref[:, 1:2]
    qz = pts_ref[:, 2:3]
    qsq = (qx * qx + qy * qy) + qz * qz
    lane = lax.broadcasted_iota(I32, (RG, 128), 1)
    bigv = jnp.full((RG, 128), BIGF, F32)
    bigi = jnp.full((RG, 128), BIGI, I32)
    outcol = lax.broadcasted_iota(I32, (RG, KNN), 1)

    def _chunk_d(c):
        cx = cand3_ref[0, pl.ds(c, 1), :]
        cy = cand3_ref[1, pl.ds(c, 1), :]
        cz = cand3_ref[2, pl.ds(c, 1), :]
        csq = (cx * cx + cy * cy) + cz * cz
        d = (qsq + csq) - 2.0 * (qx * cx + qy * cy + qz * cz)
        return d, lane + c * 128

    def _chunk_d_static(c):
        cx = cand3_ref[0, c, :].reshape(1, 128)
        cy = cand3_ref[1, c, :].reshape(1, 128)
        cz = cand3_ref[2, c, :].reshape(1, 128)
        csq = (cx * cx + cy * cy) + cz * cz
        d = (qsq + csq) - 2.0 * (qx * cx + qy * cy + qz * cz)
        return d, lane + c * 128

    def _insert(vs, ids, nv, ni):
        lt = [nv < vs[s] for s in range(NSLOT)]
        ovs = [jnp.where(lt[0], nv, vs[0])]
        ois = [jnp.where(lt[0], ni, ids[0])]
        for s in range(1, NSLOT):
            ovs.append(jnp.where(lt[s],
                                 jnp.where(lt[s - 1], vs[s - 1], nv), vs[s]))
            ois.append(jnp.where(lt[s],
                                 jnp.where(lt[s - 1], ids[s - 1], ni), ids[s]))
        return tuple(ovs), tuple(ois)

    vs, ids = tuple([bigv] * NSLOT), tuple([bigi] * NSLOT)
    for c in range(CH):
        nv0, ni0 = _chunk_d_static(c)
        vs, ids = _insert(vs, ids, nv0, ni0)

    outb = jnp.zeros((RG, KNN), I32)
    lastv = jnp.full((RG, 1), -BIGF, F32)
    lasti = jnp.full((RG, 1), -1, I32)
    for t in range(KNN):
        cv, ci = bigv, bigi
        for s in range(NSLOT):
            ok = _lex_lt(lastv, lasti, vs[s], ids[s])
            sv = jnp.where(ok, vs[s], BIGF)
            si = jnp.where(ok, ids[s], BIGI)
            t2 = _lex_lt(sv, si, cv, ci)
            cv = jnp.where(t2, sv, cv)
            ci = jnp.where(t2, si, ci)
        mv = jnp.min(cv, axis=1, keepdims=True)
        mi = jnp.min(jnp.where(cv == mv, ci, BIGI), axis=1, keepdims=True)
        outb = jnp.where(outcol == t, jnp.broadcast_to(mi, (RG, KNN)), outb)
        lastv, lasti = mv, mi

    init = (jnp.zeros((RG, KNN), I32),
            jnp.full((RG, 1), -BIGF, F32), jnp.full((RG, 1), -1, I32))

    s7v, s7i = vs[NSLOT - 1], ids[NSLOT - 1]
    bad = jnp.any((s7v < lastv) | ((s7v == lastv) & (s7i <= lasti)))

    def slow(_):
        def ext1(t, st):
            outb, lastv, lasti = st

            def scan(c, st2):
                cv, ci = st2
                d, gi = _chunk_d(c)
                ok = _lex_lt(lastv, lasti, d, gi)
                sv = jnp.where(ok, d, BIGF)
                si = jnp.where(ok, gi, BIGI)
                t2 = _lex_lt(sv, si, cv, ci)
                return jnp.where(t2, sv, cv), jnp.where(t2, si, ci)

            cv, ci = lax.fori_loop(0, CH, scan, (bigv, bigi))
            mv = jnp.min(cv, axis=1, keepdims=True)
            mi = jnp.min(jnp.where(cv == mv, ci, BIGI), axis=1, keepdims=True)
            outb = jnp.where(outcol == t, jnp.broadcast_to(mi, (RG, KNN)), outb)
            return outb, mv, mi

        outb, _, _ = lax.fori_loop(0, KNN, ext1, init)
        return outb

    outb = lax.cond(bad, slow, lambda _: outb, 0)
    row = i * RG + lax.broadcasted_iota(I32, (RG, 1), 0)
    out_ref[...] = jnp.where(row < N_REAL, outb, SENT)


def _knn_topk(cand3, pts_pad):
    return pl.pallas_call(
        _knn_body,
        grid=(NP // RG,),
        in_specs=[
            pl.BlockSpec((3, CH, 128), lambda i: (0, 0, 0)),
            pl.BlockSpec((RG, 3), lambda i: (i, 0)),
        ],
        out_specs=pl.BlockSpec((RG, KNN), lambda i: (i, 0)),
        out_shape=jax.ShapeDtypeStruct((NP, KNN), I32),
    )(cand3, pts_pad)


@functools.cache
def _sc_hist_fn():
    @functools.partial(
        pl.kernel,
        mesh=_mesh(),
        out_type=jax.ShapeDtypeStruct((NW, NP), F32),
        scratch_types=[
            pltpu.VMEM((EPW,), I32),
            pltpu.VMEM((NP,), F32),
        ],
        compiler_params=pltpu.CompilerParams(needs_layout_passes=False),
    )
    def _sc_hist(idx_hbm, out_hbm, idx_v, hist_v):
        wid = lax.axis_index("s") * NC + lax.axis_index("c")
        pltpu.sync_copy(idx_hbm.at[wid], idx_v)
        zero16 = jnp.zeros((NL,), F32)
        one16 = jnp.full((NL,), 1.0, F32)

        def zbody(j, _):
            hist_v[pl.ds(j * NL, NL)] = zero16
            return 0

        lax.fori_loop(0, NP // NL, zbody, 0)

        def abody(j, _):
            v = idx_v[pl.ds(j * NL, NL)]
            plsc.addupdate_scatter(hist_v, [v], one16)
            return 0

        lax.fori_loop(0, EPW // NL, abody, 0)
        pltpu.sync_copy(hist_v, out_hbm.at[wid])

    return _sc_hist


def _deg_body(h_ref, o_ref):
    s = jnp.sum(h_ref[...], axis=0, keepdims=True)
    o_ref[...] = lax.rsqrt(jnp.maximum(s, 1.0))


def _deg_w(hist):
    return pl.pallas_call(
        _deg_body,
        out_shape=jax.ShapeDtypeStruct((1, NP), F32),
    )(hist)


def _scale_body(x_ref, w_ref, o_ref):
    o_ref[...] = x_ref[...] * w_ref[...]


def _scale_rows(x_pad, wcol):
    return pl.pallas_call(
        _scale_body,
        grid=(NP // RB,),
        in_specs=[
            pl.BlockSpec((RB, 128), lambda i: (i, 0)),
            pl.BlockSpec((RB, 1), lambda i: (i, 0)),
        ],
        out_specs=pl.BlockSpec((RB, 128), lambda i: (i, 0)),
        out_shape=jax.ShapeDtypeStruct((NP, 128), F32),
    )(x_pad, wcol)


def _tree_sum(vs):
    while len(vs) > 1:
        nxt = [vs[i] + vs[i + 1] for i in range(0, len(vs) - 1, 2)]
        if len(vs) % 2:
            nxt.append(vs[-1])
        vs = nxt
    return vs[0]


@functools.cache
def _sc_gather_fn():
    @functools.partial(
        pl.kernel,
        mesh=_mesh(),
        out_type=jax.ShapeDtypeStruct((NP, 128), F32),
        scratch_types=[
            pltpu.VMEM((NB + 1, BN * KNN), I32),
            pltpu.VMEM((BN * KNN, 128), F32),
            pltpu.VMEM((BN * KNN, 128), F32),
            pltpu.VMEM((BN, 128), F32),
            pltpu.VMEM((BN, 128), F32),
            pltpu.SemaphoreType.DMA,
            pltpu.SemaphoreType.DMA,
            pltpu.SemaphoreType.DMA,
            pltpu.SemaphoreType.DMA,
        ],
        compiler_params=pltpu.CompilerParams(needs_layout_passes=False),
    )
    def _sc_gather(hs_hbm, idx_hbm, out_hbm, idx_v, rows_a, rows_b,
                   out_a, out_b, sem_a, sem_b, sem_oa, sem_ob):
        wid = lax.axis_index("s") * NC + lax.axis_index("c")
        pltpu.sync_copy(idx_hbm.at[wid], idx_v.at[pl.ds(0, NB)])
        zero16 = jnp.zeros((NL,), I32)
        for k in range(BN * KNN // NL):
            idx_v[NB, pl.ds(k * NL, NL)] = zero16
        pltpu.make_async_copy(hs_hbm.at[idx_v.at[0]], rows_a, sem_a).start()

        def _sum_into(rows, outbuf):
            for n in range(BN):
                for c in range(8):
                    sl = pl.ds(c * NL, NL)
                    acc = _tree_sum([rows[n * KNN + m, sl] for m in range(KNN)])
                    outbuf[n, sl] = acc

        def pair(j, _):
            b = 2 * j
            base = wid * NPW + b * BN
            pltpu.make_async_copy(hs_hbm.at[idx_v.at[b + 1]], rows_b, sem_b).start()
            pltpu.make_async_copy(hs_hbm.at[idx_v.at[b]], rows_a, sem_a).wait()

            @pl.when(j >= 1)
            def _():
                pltpu.make_async_copy(out_a, out_hbm.at[pl.ds(0, BN)], sem_oa).wait()

            _sum_into(rows_a, out_a)
            pltpu.make_async_copy(hs_hbm.at[idx_v.at[b + 2]], rows_a, sem_a).start()
            pltpu.make_async_copy(out_a, out_hbm.at[pl.ds(base, BN)], sem_oa).start()
            pltpu.make_async_copy(hs_hbm.at[idx_v.at[b + 1]], rows_b, sem_b).wait()

            @pl.when(j >= 1)
            def _():
                pltpu.make_async_copy(out_b, out_hbm.at[pl.ds(0, BN)], sem_ob).wait()

            _sum_into(rows_b, out_b)
            pltpu.make_async_copy(out_b, out_hbm.at[pl.ds(base + BN, BN)], sem_ob).start()
            return 0

        lax.fori_loop(0, NB // 2, pair, 0)
        pltpu.make_async_copy(out_a, out_hbm.at[pl.ds(0, BN)], sem_oa).wait()
        pltpu.make_async_copy(out_b, out_hbm.at[pl.ds(0, BN)], sem_ob).wait()
        pltpu.make_async_copy(hs_hbm.at[idx_v.at[NB]], rows_a, sem_a).wait()

    return _sc_gather


def _mm_body(a_ref, w_ref, b_ref, wc_ref, o_ref):
    a = a_ref[...] * INV_SQRT_K
    m = lax.dot_general(a, w_ref[...], (((1,), (0,)), ((), ())),
                        precision=lax.Precision.HIGHEST,
                        preferred_element_type=F32)
    h = jnp.maximum(m + b_ref[...], 0.0)
    o_ref[...] = h * wc_ref[...]


def _mm_layer(agg, W, b, wcol):
    return pl.pallas_call(
        _mm_body,
        grid=(NP // RB,),
        in_specs=[
            pl.BlockSpec((RB, 128), lambda i: (i, 0)),
            pl.BlockSpec((128, 128), lambda i: (0, 0)),
            pl.BlockSpec((1, 128), lambda i: (0, 0)),
            pl.BlockSpec((RB, 1), lambda i: (i, 0)),
        ],
        out_specs=pl.BlockSpec((RB, 128), lambda i: (i, 0)),
        out_shape=jax.ShapeDtypeStruct((NP, 128), F32),
    )(agg, W, b, wcol)


def _mm_final_body(a_ref, w_ref, b_ref, wf_ref, bf_ref, o_ref):
    a = a_ref[...] * INV_SQRT_K
    m = lax.dot_general(a, w_ref[...], (((1,), (0,)), ((), ())),
                        precision=lax.Precision.HIGHEST,
                        preferred_element_type=F32)
    h = jnp.maximum(m + b_ref[...], 0.0)
    z = lax.dot_general(h, wf_ref[...], (((1,), (0,)), ((), ())),
                        precision=lax.Precision.HIGHEST,
                        preferred_element_type=F32) + bf_ref[...]
    o_ref[...] = 1.0 / (1.0 + jnp.exp(-z))


def _mm_final(agg, W, b, Wf, bf):
    return pl.pallas_call(
        _mm_final_body,
        grid=(NP // RB,),
        in_specs=[
            pl.BlockSpec((RB, 128), lambda i: (i, 0)),
            pl.BlockSpec((128, 128), lambda i: (0, 0)),
            pl.BlockSpec((1, 128), lambda i: (0, 0)),
            pl.BlockSpec((128, 1), lambda i: (0, 0)),
            pl.BlockSpec((1, 1), lambda i: (0, 0)),
        ],
        out_specs=pl.BlockSpec((RB, 1), lambda i: (i, 0)),
        out_shape=jax.ShapeDtypeStruct((NP, 1), F32),
    )(agg, W, b, Wf, bf)


def kernel(x, triangle_centers, W1, b1, W2, b2, W3, b3, Wf, bf):
    pts = triangle_centers
    pad_n = NP - N_REAL
    pts_pad = jnp.pad(pts, ((0, pad_n), (0, 0)), constant_values=PADC)
    cand3 = jnp.pad(pts.T, ((0, 0), (0, pad_n)),
                    constant_values=PADC).reshape(3, CH, 128)
    x_pad = jnp.pad(x, ((0, pad_n), (0, 0)))

    idx = _knn_topk(cand3, pts_pad)
    hist = _sc_hist_fn()(idx.reshape(NW, EPW))
    w1d = _deg_w(hist)
    wcol = w1d.reshape(NP, 1)

    idx3 = idx.reshape(NW, NB, BN * KNN)
    h = _scale_rows(x_pad, wcol)
    for W, b in ((W1, b1), (W2, b2)):
        agg = _sc_gather_fn()(h, idx3)
        h = _mm_layer(agg, W, b.reshape(1, 128), wcol)
    agg = _sc_gather_fn()(h, idx3)
    res = _mm_final(agg, W3, b3.reshape(1, 128), Wf, bf.reshape(1, 1))
    return res[:N_REAL, 0]
